# Initial kernel scaffold; baseline (speedup 1.0000x reference)
import jax
import jax.numpy as jnp
from jax.experimental import pallas as pl


def kernel(x, edge_index, batch_index, W1, b1, W2, b2, Wf1, bf1, Wf2, bf2):
    raise NotImplementedError("write your pallas kernel here")


# speedup vs baseline: 36.8464x; 36.8464x over previous
import functools

import jax
import jax.numpy as jnp
from jax import lax
from jax.experimental import pallas as pl
from jax.experimental.pallas import tpu as pltpu
from jax.experimental.pallas import tpu_sc as plsc

_N = 10000
_E = 320000
_DIN = 128
_DH = 32
_DOUT = 2
_G = 64
_DFC = 128

_NC = 2
_NS = 16
_NW = _NC * _NS

_C = 125
_K = 8
_EROWS = _E // _C
_RPT = _EROWS // _NW
_NOUT = _RPT // _K

_NP = 10240
_RNODE = _NP // _NS

_mesh = plsc.VectorSubcoreMesh(core_axis_name="c", subcore_axis_name="s")


@functools.partial(
    pl.kernel,
    out_type=jax.ShapeDtypeStruct((_NC * _NP,), jnp.float32),
    mesh=_mesh,
    scratch_types=[
        pltpu.VMEM((_K, _C), jnp.int32),
        pltpu.VMEM((128,), jnp.float32),
        pltpu.VMEM_SHARED((_NP,), jnp.float32),
    ],
    compiler_params=pltpu.CompilerParams(use_tc_tiling_on_sc=False),
)
def _sc_degree(dst2d, zeros_n, out, didx, ones_v, acc):
    c = lax.axis_index("c")
    s = lax.axis_index("s")
    wid = s * _NC + c

    for j in range(128 // 16):
        ones_v[pl.ds(j * 16, 16)] = jnp.ones((16,), jnp.float32)

    pltpu.sync_copy(zeros_n.at[pl.ds(s * _RNODE, _RNODE)],
                    acc.at[pl.ds(s * _RNODE, _RNODE)])
    plsc.subcore_barrier()

    base = wid * _RPT

    def body(i, carry):
        r0 = base + i * _K
        pltpu.sync_copy(dst2d.at[pl.ds(r0, _K), :], didx)
        for j in range(_K):
            pltpu.sync_copy(ones_v.at[pl.ds(0, _C)], acc.at[didx.at[j]],
                            add=True)
        return carry

    lax.fori_loop(0, _NOUT, body, 0)

    plsc.subcore_barrier()
    pltpu.sync_copy(acc.at[pl.ds(s * _RNODE, _RNODE)],
                    out.at[pl.ds(c * _NP + s * _RNODE, _RNODE)])


@functools.partial(
    pl.kernel,
    out_type=jax.ShapeDtypeStruct((_NC * _NP, _DH), jnp.float32),
    mesh=_mesh,
    scratch_types=[
        pltpu.VMEM((_K, _C), jnp.int32),
        pltpu.VMEM((_K, _C), jnp.int32),
        pltpu.VMEM((_K, _C, _DH), jnp.float32),
        pltpu.VMEM_SHARED((_NP, _DH), jnp.float32),
        pltpu.SemaphoreType.DMA,
    ],
    compiler_params=pltpu.CompilerParams(use_tc_tiling_on_sc=False),
)
def _sc_message(src2d, dst2d, table, zeros_t, out, sidx, didx, rows, acc, sem):
    c = lax.axis_index("c")
    s = lax.axis_index("s")
    wid = s * _NC + c

    pltpu.sync_copy(zeros_t.at[pl.ds(s * _RNODE, _RNODE), :],
                    acc.at[pl.ds(s * _RNODE, _RNODE), :])
    plsc.subcore_barrier()

    base = wid * _RPT

    def body(i, carry):
        r0 = base + i * _K
        pltpu.sync_copy(src2d.at[pl.ds(r0, _K), :], sidx)
        pltpu.sync_copy(dst2d.at[pl.ds(r0, _K), :], didx)
        cps = [pltpu.async_copy(table.at[sidx.at[j]], rows.at[j], sem)
               for j in range(_K)]
        for cp in cps:
            cp.wait()
        for j in range(_K):
            pltpu.sync_copy(rows.at[j], acc.at[didx.at[j]], add=True)
        return carry

    lax.fori_loop(0, _NOUT, body, 0)

    plsc.subcore_barrier()
    pltpu.sync_copy(acc.at[pl.ds(s * _RNODE, _RNODE), :],
                    out.at[pl.ds(c * _NP + s * _RNODE, _RNODE), :])


_BLK = 640
_GRID = _NP // _BLK


def _tc1_body(degp_ref, x_ref, w1_ref, xwp_ref, dinv_ref):
    deg = degp_ref[:, 0:1] + degp_ref[:, 1:2] + 1.0
    dinv = lax.rsqrt(deg)
    xw = jnp.dot(x_ref[...], w1_ref[...], precision=lax.Precision.HIGHEST)
    xwp_ref[...] = dinv * xw
    dinv_ref[...] = dinv


def _tc1(degp, xp, W1):
    return pl.pallas_call(
        _tc1_body,
        grid=(_GRID,),
        in_specs=[
            pl.BlockSpec((_BLK, 2), lambda i: (i, 0)),
            pl.BlockSpec((_BLK, _DIN), lambda i: (i, 0)),
            pl.BlockSpec((_DIN, _DH), lambda i: (0, 0)),
        ],
        out_specs=[
            pl.BlockSpec((_BLK, _DH), lambda i: (i, 0)),
            pl.BlockSpec((_BLK, 1), lambda i: (i, 0)),
        ],
        out_shape=[
            jax.ShapeDtypeStruct((_NP, _DH), jnp.float32),
            jax.ShapeDtypeStruct((_NP, 1), jnp.float32),
        ],
    )(degp, xp, W1)


def _tc2_body(a0_ref, a1_ref, xwp_ref, dinv_ref, b1_ref, w2_ref, out_ref):
    dinv = dinv_ref[...]
    h = dinv * (a0_ref[...] + a1_ref[...] + xwp_ref[...]) + b1_ref[...]
    h = jnp.maximum(h, 0.0)
    hw = jnp.dot(h, w2_ref[...], precision=lax.Precision.HIGHEST)
    out_ref[...] = dinv * hw


def _tc2(a0, a1, xwp, dinv, b1, W2):
    return pl.pallas_call(
        _tc2_body,
        grid=(_GRID,),
        in_specs=[
            pl.BlockSpec((_BLK, _DH), lambda i: (i, 0)),
            pl.BlockSpec((_BLK, _DH), lambda i: (i, 0)),
            pl.BlockSpec((_BLK, _DH), lambda i: (i, 0)),
            pl.BlockSpec((_BLK, 1), lambda i: (i, 0)),
            pl.BlockSpec((1, _DH), lambda i: (0, 0)),
            pl.BlockSpec((_DH, _DH), lambda i: (0, 0)),
        ],
        out_specs=pl.BlockSpec((_BLK, _DH), lambda i: (i, 0)),
        out_shape=jax.ShapeDtypeStruct((_NP, _DH), jnp.float32),
    )(a0, a1, xwp, dinv, b1, W2)


def _tc3_body(a0_ref, a1_ref, xwp_ref, dinv_ref, b2_ref, bi_ref,
              wf1_ref, bf1_ref, wf2_ref, bf2_ref, out_ref):
    h = dinv_ref[...] * (a0_ref[...] + a1_ref[...] + xwp_ref[...]) + b2_ref[...]
    h = jnp.maximum(h, 0.0)
    gids = lax.broadcasted_iota(jnp.int32, (_NP, _G), 1)
    onehot = (bi_ref[...] == gids).astype(jnp.float32)
    dn = (((0,), (0,)), ((), ()))
    ssum = lax.dot_general(onehot, h, dn, precision=lax.Precision.HIGHEST)
    cnt = lax.dot_general(onehot, jnp.ones((_NP, 1), jnp.float32), dn,
                          precision=lax.Precision.HIGHEST)
    gemb = ssum / jnp.maximum(cnt, 1.0)
    z = jnp.dot(gemb, wf1_ref[...], precision=lax.Precision.HIGHEST) + bf1_ref[...]
    z = jnp.maximum(z, 0.0)
    out_ref[...] = jnp.dot(z, wf2_ref[...],
                           precision=lax.Precision.HIGHEST) + bf2_ref[...]


def _tc3(a0, a1, xwp, dinv, b2, bi2d, Wf1, bf1, Wf2, bf2):
    return pl.pallas_call(
        _tc3_body,
        out_shape=jax.ShapeDtypeStruct((_G, _DOUT), jnp.float32),
    )(a0, a1, xwp, dinv, b2, bi2d, Wf1, bf1, Wf2, bf2)


@jax.jit
def kernel(x, edge_index, batch_index, W1, b1, W2, b2, Wf1, bf1, Wf2, bf2):
    f32 = jnp.float32
    src2d = edge_index[0].reshape(_EROWS, _C)
    dst2d = edge_index[1].reshape(_EROWS, _C)

    pad = _NP - _N
    xp = jnp.concatenate([x, jnp.zeros((pad, _DIN), f32)], axis=0)
    bip = jnp.concatenate(
        [batch_index, jnp.full((pad,), _G, batch_index.dtype)]).reshape(_NP, 1)
    zeros_n = jnp.zeros((_NP,), f32)
    zeros_t = jnp.zeros((_NP, _DH), f32)

    degp = _sc_degree(dst2d, zeros_n)
    degp = degp.reshape(_NC, _NP).T

    xwp1, dinv = _tc1(degp, xp, W1)

    acc1 = _sc_message(src2d, dst2d, xwp1, zeros_t)
    xwp2 = _tc2(acc1[:_NP], acc1[_NP:], xwp1, dinv, b1.reshape(1, _DH), W2)

    acc2 = _sc_message(src2d, dst2d, xwp2, zeros_t)
    out = _tc3(acc2[:_NP], acc2[_NP:], xwp2, dinv, b2.reshape(1, _DH), bip,
               Wf1, bf1.reshape(1, _DFC), Wf2, bf2.reshape(1, _DOUT))
    return out


# speedup vs baseline: 43.8249x; 1.1894x over previous
import functools

import jax
import jax.numpy as jnp
from jax import lax
from jax.experimental import pallas as pl
from jax.experimental.pallas import tpu as pltpu
from jax.experimental.pallas import tpu_sc as plsc

_N = 10000
_E = 320000
_DIN = 128
_DH = 32
_DOUT = 2
_G = 64
_DFC = 128

_NC = 2
_NS = 16
_NW = _NC * _NS

_C = 125
_K = 8
_EROWS = _E // _C
_RPT = _EROWS // _NW
_NOUT = _RPT // _K

_NP = 10240
_RNODE = _NP // _NS

_mesh = plsc.VectorSubcoreMesh(core_axis_name="c", subcore_axis_name="s")


@functools.partial(
    pl.kernel,
    out_type=jax.ShapeDtypeStruct((_NC * _NP,), jnp.float32),
    mesh=_mesh,
    scratch_types=[
        pltpu.VMEM((_K, _C), jnp.int32),
        pltpu.VMEM((128,), jnp.float32),
        pltpu.VMEM_SHARED((_NP,), jnp.float32),
    ],
    compiler_params=pltpu.CompilerParams(use_tc_tiling_on_sc=False),
)
def _sc_degree(dst2d, zeros_n, out, didx, ones_v, acc):
    c = lax.axis_index("c")
    s = lax.axis_index("s")
    wid = s * _NC + c

    for j in range(128 // 16):
        ones_v[pl.ds(j * 16, 16)] = jnp.ones((16,), jnp.float32)

    pltpu.sync_copy(zeros_n.at[pl.ds(s * _RNODE, _RNODE)],
                    acc.at[pl.ds(s * _RNODE, _RNODE)])
    plsc.subcore_barrier()

    base = wid * _RPT

    def body(i, carry):
        r0 = base + i * _K
        pltpu.sync_copy(dst2d.at[pl.ds(r0, _K), :], didx)
        for j in range(_K):
            pltpu.sync_copy(ones_v.at[pl.ds(0, _C)], acc.at[didx.at[j]],
                            add=True)
        return carry

    lax.fori_loop(0, _NOUT, body, 0)

    plsc.subcore_barrier()
    pltpu.sync_copy(acc.at[pl.ds(s * _RNODE, _RNODE)],
                    out.at[pl.ds(c * _NP + s * _RNODE, _RNODE)])


@functools.partial(
    pl.kernel,
    out_type=jax.ShapeDtypeStruct((_NC * _NP, _DH), jnp.float32),
    mesh=_mesh,
    scratch_types=[
        [pltpu.VMEM((_K, 2, _C), jnp.int32)] * 3,
        [pltpu.VMEM((_K, _C, _DH), jnp.float32)] * 2,
        pltpu.VMEM_SHARED((_NP, _DH), jnp.float32),
        pltpu.SemaphoreType.DMA,
        pltpu.SemaphoreType.DMA,
        pltpu.SemaphoreType.DMA,
    ],
    compiler_params=pltpu.CompilerParams(use_tc_tiling_on_sc=False),
)
def _sc_message(ei3d, table, zeros_t, out, ibuf, rows, acc,
                sem_i, sem_g, sem_s):
    c = lax.axis_index("c")
    s = lax.axis_index("s")
    wid = s * _NC + c

    pltpu.sync_copy(zeros_t.at[pl.ds(s * _RNODE, _RNODE), :],
                    acc.at[pl.ds(s * _RNODE, _RNODE), :])
    plsc.subcore_barrier()

    base = wid * _RPT

    idx_d = [None] * (_NOUT + 1)
    sc_d = [None] * _NOUT
    idx_d[0] = pltpu.async_copy(ei3d.at[pl.ds(base, _K)], ibuf[0], sem_i)
    for i in range(_NOUT):
        rb = rows[i % 2]
        ib = ibuf[i % 3]
        if i >= 2:
            for d in sc_d[i - 2]:
                d.wait()
        idx_d[i].wait()
        if i + 1 < _NOUT:
            idx_d[i + 1] = pltpu.async_copy(
                ei3d.at[pl.ds(base + (i + 1) * _K, _K)],
                ibuf[(i + 1) % 3], sem_i)
        gs = [pltpu.async_copy(table.at[ib.at[j, 0]], rb.at[j], sem_g)
              for j in range(_K)]
        for g in gs:
            g.wait()
        sc_d[i] = [pltpu.async_copy(rb.at[j], acc.at[ib.at[j, 1]], sem_s,
                                    add=True)
                   for j in range(_K)]
    for i in (_NOUT - 2, _NOUT - 1):
        for d in sc_d[i]:
            d.wait()

    plsc.subcore_barrier()
    pltpu.sync_copy(acc.at[pl.ds(s * _RNODE, _RNODE), :],
                    out.at[pl.ds(c * _NP + s * _RNODE, _RNODE), :])


_BLK = 640
_GRID = _NP // _BLK


def _tc1_body(degp_ref, x_ref, w1_ref, xwp_ref, dinv_ref):
    deg = degp_ref[:, 0:1] + degp_ref[:, 1:2] + 1.0
    dinv = lax.rsqrt(deg)
    xw = jnp.dot(x_ref[...], w1_ref[...])
    xwp_ref[...] = dinv * xw
    dinv_ref[...] = dinv


def _tc1(degp, xp, W1):
    return pl.pallas_call(
        _tc1_body,
        grid=(_GRID,),
        in_specs=[
            pl.BlockSpec((_BLK, 2), lambda i: (i, 0)),
            pl.BlockSpec((_BLK, _DIN), lambda i: (i, 0)),
            pl.BlockSpec((_DIN, _DH), lambda i: (0, 0)),
        ],
        out_specs=[
            pl.BlockSpec((_BLK, _DH), lambda i: (i, 0)),
            pl.BlockSpec((_BLK, 1), lambda i: (i, 0)),
        ],
        out_shape=[
            jax.ShapeDtypeStruct((_NP, _DH), jnp.float32),
            jax.ShapeDtypeStruct((_NP, 1), jnp.float32),
        ],
    )(degp, xp, W1)


def _tc2_body(a0_ref, a1_ref, xwp_ref, dinv_ref, b1_ref, w2_ref, out_ref):
    dinv = dinv_ref[...]
    h = dinv * (a0_ref[...] + a1_ref[...] + xwp_ref[...]) + b1_ref[...]
    h = jnp.maximum(h, 0.0)
    hw = jnp.dot(h, w2_ref[...])
    out_ref[...] = dinv * hw


def _tc2(a0, a1, xwp, dinv, b1, W2):
    return pl.pallas_call(
        _tc2_body,
        grid=(_GRID,),
        in_specs=[
            pl.BlockSpec((_BLK, _DH), lambda i: (i, 0)),
            pl.BlockSpec((_BLK, _DH), lambda i: (i, 0)),
            pl.BlockSpec((_BLK, _DH), lambda i: (i, 0)),
            pl.BlockSpec((_BLK, 1), lambda i: (i, 0)),
            pl.BlockSpec((1, _DH), lambda i: (0, 0)),
            pl.BlockSpec((_DH, _DH), lambda i: (0, 0)),
        ],
        out_specs=pl.BlockSpec((_BLK, _DH), lambda i: (i, 0)),
        out_shape=jax.ShapeDtypeStruct((_NP, _DH), jnp.float32),
    )(a0, a1, xwp, dinv, b1, W2)


def _tc3_body(a0_ref, a1_ref, xwp_ref, dinv_ref, b2_ref, bi_ref,
              wf1_ref, bf1_ref, wf2_ref, bf2_ref, out_ref):
    h = dinv_ref[...] * (a0_ref[...] + a1_ref[...] + xwp_ref[...]) + b2_ref[...]
    h = jnp.maximum(h, 0.0)
    gids = lax.broadcasted_iota(jnp.int32, (_NP, _G), 1)
    onehot = (bi_ref[...] == gids).astype(jnp.float32)
    dn = (((0,), (0,)), ((), ()))
    ssum = lax.dot_general(onehot, h, dn, precision=lax.Precision.HIGHEST)
    cnt = lax.dot_general(onehot, jnp.ones((_NP, 1), jnp.float32), dn,
                          precision=lax.Precision.HIGHEST)
    gemb = ssum / jnp.maximum(cnt, 1.0)
    z = jnp.dot(gemb, wf1_ref[...]) + bf1_ref[...]
    z = jnp.maximum(z, 0.0)
    out_ref[...] = jnp.dot(z, wf2_ref[...]) + bf2_ref[...]


def _tc3(a0, a1, xwp, dinv, b2, bi2d, Wf1, bf1, Wf2, bf2):
    return pl.pallas_call(
        _tc3_body,
        out_shape=jax.ShapeDtypeStruct((_G, _DOUT), jnp.float32),
    )(a0, a1, xwp, dinv, b2, bi2d, Wf1, bf1, Wf2, bf2)


@jax.jit
def kernel(x, edge_index, batch_index, W1, b1, W2, b2, Wf1, bf1, Wf2, bf2):
    f32 = jnp.float32
    src2d = edge_index[0].reshape(_EROWS, _C)
    dst2d = edge_index[1].reshape(_EROWS, _C)
    ei3d = jnp.stack([src2d, dst2d], axis=1)

    pad = _NP - _N
    xp = jnp.concatenate([x, jnp.zeros((pad, _DIN), f32)], axis=0)
    bip = jnp.concatenate(
        [batch_index, jnp.full((pad,), _G, batch_index.dtype)]).reshape(_NP, 1)
    zeros_n = jnp.zeros((_NP,), f32)
    zeros_t = jnp.zeros((_NP, _DH), f32)

    degp = _sc_degree(dst2d, zeros_n)
    degp = degp.reshape(_NC, _NP).T

    xwp1, dinv = _tc1(degp, xp, W1)

    acc1 = _sc_message(ei3d, xwp1, zeros_t)
    xwp2 = _tc2(acc1[:_NP], acc1[_NP:], xwp1, dinv, b1.reshape(1, _DH), W2)

    acc2 = _sc_message(ei3d, xwp2, zeros_t)
    out = _tc3(acc2[:_NP], acc2[_NP:], xwp2, dinv, b2.reshape(1, _DH), bip,
               Wf1, bf1.reshape(1, _DFC), Wf2, bf2.reshape(1, _DOUT))
    return out


# speedup vs baseline: 49.7913x; 1.1361x over previous
import functools

import jax
import jax.numpy as jnp
from jax import lax
from jax.experimental import pallas as pl
from jax.experimental.pallas import tpu as pltpu
from jax.experimental.pallas import tpu_sc as plsc

_N = 10000
_E = 320000
_DIN = 128
_DH = 32
_DOUT = 2
_G = 64
_DFC = 128

_NC = 2
_NS = 16
_NW = _NC * _NS

_C = 125
_K = 8
_EROWS = _E // _C
_RPT = _EROWS // _NW
_NOUT = _RPT // _K

_NP = 10240
_RNODE = _NP // _NS

_mesh = plsc.VectorSubcoreMesh(core_axis_name="c", subcore_axis_name="s")


@functools.partial(
    pl.kernel,
    out_type=jax.ShapeDtypeStruct((_NC * _NP,), jnp.float32),
    mesh=_mesh,
    scratch_types=[
        pltpu.VMEM((_K, _C), jnp.int32),
        pltpu.VMEM((_RNODE,), jnp.float32),
        pltpu.VMEM_SHARED((_NP,), jnp.float32),
    ],
    compiler_params=pltpu.CompilerParams(use_tc_tiling_on_sc=False),
)
def _sc_degree(dst2d, out, didx, ones_v, acc):
    c = lax.axis_index("c")
    s = lax.axis_index("s")
    wid = s * _NC + c

    def zfill(j, carry):
        ones_v[pl.ds(j * 16, 16)] = jnp.zeros((16,), jnp.float32)
        return carry

    lax.fori_loop(0, _RNODE // 16, zfill, 0)
    pltpu.sync_copy(ones_v, acc.at[pl.ds(s * _RNODE, _RNODE)])
    for j in range(128 // 16):
        ones_v[pl.ds(j * 16, 16)] = jnp.ones((16,), jnp.float32)
    plsc.subcore_barrier()

    base = wid * _RPT

    def body(i, carry):
        r0 = base + i * _K
        pltpu.sync_copy(dst2d.at[pl.ds(r0, _K), :], didx)
        for j in range(_K):
            pltpu.sync_copy(ones_v.at[pl.ds(0, _C)], acc.at[didx.at[j]],
                            add=True)
        return carry

    lax.fori_loop(0, _NOUT, body, 0)

    plsc.subcore_barrier()
    pltpu.sync_copy(acc.at[pl.ds(s * _RNODE, _RNODE)],
                    out.at[pl.ds(c * _NP + s * _RNODE, _RNODE)])


@functools.partial(
    pl.kernel,
    out_type=jax.ShapeDtypeStruct((_NC * _NP, _DH), jnp.float32),
    mesh=_mesh,
    scratch_types=[
        [pltpu.VMEM((2, _K, _C), jnp.int32)] * 3,
        [pltpu.VMEM((_K, _C, _DH), jnp.float32)] * 2,
        pltpu.VMEM((80, _DH), jnp.float32),
        pltpu.VMEM_SHARED((_NP, _DH), jnp.float32),
        pltpu.SemaphoreType.DMA,
        pltpu.SemaphoreType.DMA,
        pltpu.SemaphoreType.DMA,
    ],
    compiler_params=pltpu.CompilerParams(use_tc_tiling_on_sc=False),
)
def _sc_message(ei3d, table, out, ibuf, rows, zbuf, acc,
                sem_i, sem_g, sem_s):
    c = lax.axis_index("c")
    s = lax.axis_index("s")
    wid = s * _NC + c

    def zfill(k, carry):
        zbuf[k, pl.ds(0, 16)] = jnp.zeros((16,), jnp.float32)
        zbuf[k, pl.ds(16, 16)] = jnp.zeros((16,), jnp.float32)
        return carry

    lax.fori_loop(0, 80, zfill, 0)
    for t in range(_RNODE // 80):
        pltpu.sync_copy(zbuf, acc.at[pl.ds(s * _RNODE + t * 80, 80), :])
    plsc.subcore_barrier()

    base = wid * _RPT

    idx_d = [None] * (_NOUT + 1)
    sc_d = [None] * _NOUT
    idx_d[0] = pltpu.async_copy(ei3d.at[:, pl.ds(base, _K), :], ibuf[0],
                                sem_i)
    for i in range(_NOUT):
        rb = rows[i % 2]
        ib = ibuf[i % 3]
        if i >= 2:
            for d in sc_d[i - 2]:
                d.wait()
        idx_d[i].wait()
        if i + 1 < _NOUT:
            idx_d[i + 1] = pltpu.async_copy(
                ei3d.at[:, pl.ds(base + (i + 1) * _K, _K), :],
                ibuf[(i + 1) % 3], sem_i)
        gs = [pltpu.async_copy(table.at[ib.at[0, j]], rb.at[j], sem_g)
              for j in range(_K)]
        for g in gs:
            g.wait()
        sc_d[i] = [pltpu.async_copy(rb.at[j], acc.at[ib.at[1, j]], sem_s,
                                    add=True)
                   for j in range(_K)]
    for i in (_NOUT - 2, _NOUT - 1):
        for d in sc_d[i]:
            d.wait()

    plsc.subcore_barrier()
    pltpu.sync_copy(acc.at[pl.ds(s * _RNODE, _RNODE), :],
                    out.at[pl.ds(c * _NP + s * _RNODE, _RNODE), :])


_BLK = 2048
_GRID = _NP // _BLK


def _tc1_body(d0_ref, d1_ref, x_ref, w1_ref, xwp_ref, dinv_ref):
    deg = d0_ref[...] + d1_ref[...] + 1.0
    dinv = lax.rsqrt(deg)
    dinv_ref[...] = dinv
    xw = jnp.dot(x_ref[...], w1_ref[...])
    xwp_ref[...] = dinv[:, None] * xw


def _tc1(d0, d1, xp, W1):
    return pl.pallas_call(
        _tc1_body,
        grid=(_GRID,),
        in_specs=[
            pl.BlockSpec((_BLK,), lambda i: (i,)),
            pl.BlockSpec((_BLK,), lambda i: (i,)),
            pl.BlockSpec((_BLK, _DIN), lambda i: (i, 0)),
            pl.BlockSpec((_DIN, _DH), lambda i: (0, 0)),
        ],
        out_specs=[
            pl.BlockSpec((_BLK, _DH), lambda i: (i, 0)),
            pl.BlockSpec((_BLK,), lambda i: (i,)),
        ],
        out_shape=[
            jax.ShapeDtypeStruct((_NP, _DH), jnp.float32),
            jax.ShapeDtypeStruct((_NP,), jnp.float32),
        ],
    )(d0, d1, xp, W1)


def _tc2_body(a0_ref, a1_ref, xwp_ref, dinv_ref, b1_ref, w2_ref, out_ref):
    dinv = dinv_ref[...][:, None]
    h = dinv * (a0_ref[...] + a1_ref[...] + xwp_ref[...]) + b1_ref[...]
    h = jnp.maximum(h, 0.0)
    hw = jnp.dot(h, w2_ref[...])
    out_ref[...] = dinv * hw


def _tc2(a0, a1, xwp, dinv, b1, W2):
    return pl.pallas_call(
        _tc2_body,
        grid=(_GRID,),
        in_specs=[
            pl.BlockSpec((_BLK, _DH), lambda i: (i, 0)),
            pl.BlockSpec((_BLK, _DH), lambda i: (i, 0)),
            pl.BlockSpec((_BLK, _DH), lambda i: (i, 0)),
            pl.BlockSpec((_BLK,), lambda i: (i,)),
            pl.BlockSpec((1, _DH), lambda i: (0, 0)),
            pl.BlockSpec((_DH, _DH), lambda i: (0, 0)),
        ],
        out_specs=pl.BlockSpec((_BLK, _DH), lambda i: (i, 0)),
        out_shape=jax.ShapeDtypeStruct((_NP, _DH), jnp.float32),
    )(a0, a1, xwp, dinv, b1, W2)


def _tc3_body(a0_ref, a1_ref, xwp_ref, dinv_ref, b2_ref, bi_ref,
              wf1_ref, bf1_ref, wf2_ref, bf2_ref, out_ref):
    h = (dinv_ref[...][:, None] * (a0_ref[...] + a1_ref[...] + xwp_ref[...])
         + b2_ref[...])
    h = jnp.maximum(h, 0.0)
    gids = lax.broadcasted_iota(jnp.int32, (_NP, _G), 1)
    onehot = (bi_ref[...] == gids).astype(jnp.float32)
    dn = (((0,), (0,)), ((), ()))
    ssum = lax.dot_general(onehot, h, dn, precision=lax.Precision.HIGHEST)
    cnt = lax.dot_general(onehot, jnp.ones((_NP, 1), jnp.float32), dn,
                          precision=lax.Precision.HIGHEST)
    gemb = ssum / jnp.maximum(cnt, 1.0)
    z = jnp.dot(gemb, wf1_ref[...]) + bf1_ref[...]
    z = jnp.maximum(z, 0.0)
    out_ref[...] = jnp.dot(z, wf2_ref[...]) + bf2_ref[...]


def _tc3(a0, a1, xwp, dinv, b2, bi2d, Wf1, bf1, Wf2, bf2):
    return pl.pallas_call(
        _tc3_body,
        out_shape=jax.ShapeDtypeStruct((_G, _DOUT), jnp.float32),
    )(a0, a1, xwp, dinv, b2, bi2d, Wf1, bf1, Wf2, bf2)


@jax.jit
def kernel(x, edge_index, batch_index, W1, b1, W2, b2, Wf1, bf1, Wf2, bf2):
    f32 = jnp.float32
    dst2d = edge_index[1].reshape(_EROWS, _C)
    ei3d = edge_index.reshape(2, _EROWS, _C)

    pad = _NP - _N
    xp = jnp.concatenate([x, jnp.zeros((pad, _DIN), f32)], axis=0)
    bip = jnp.concatenate(
        [batch_index, jnp.full((pad,), _G, batch_index.dtype)]).reshape(_NP, 1)

    degp = _sc_degree(dst2d)

    xwp1, dinv = _tc1(degp[:_NP], degp[_NP:], xp, W1)

    acc1 = _sc_message(ei3d, xwp1)
    xwp2 = _tc2(acc1[:_NP], acc1[_NP:], xwp1, dinv, b1.reshape(1, _DH), W2)

    acc2 = _sc_message(ei3d, xwp2)
    out = _tc3(acc2[:_NP], acc2[_NP:], xwp2, dinv, b2.reshape(1, _DH), bip,
               Wf1, bf1.reshape(1, _DFC), Wf2, bf2.reshape(1, _DOUT))
    return out


# speedup vs baseline: 52.2988x; 1.0504x over previous
import functools

import jax
import jax.numpy as jnp
from jax import lax
from jax.experimental import pallas as pl
from jax.experimental.pallas import tpu as pltpu
from jax.experimental.pallas import tpu_sc as plsc

_N = 10000
_E = 320000
_DIN = 128
_DH = 32
_DOUT = 2
_G = 64
_DFC = 128

_NC = 2
_NS = 16
_NW = _NC * _NS

_C = 125
_K = 8
_EROWS = _E // _C
_RPT = _EROWS // _NW
_NOUT = _RPT // _K

_NP = 10240
_RNODE = _NP // _NS

_mesh = plsc.VectorSubcoreMesh(core_axis_name="c", subcore_axis_name="s")


@functools.partial(
    pl.kernel,
    out_type=jax.ShapeDtypeStruct((_NC * _NP,), jnp.float32),
    mesh=_mesh,
    scratch_types=[
        [pltpu.VMEM((_K, _C), jnp.int32)] * 3,
        pltpu.VMEM((_RNODE,), jnp.float32),
        pltpu.VMEM_SHARED((_NP,), jnp.float32),
        pltpu.SemaphoreType.DMA,
        pltpu.SemaphoreType.DMA,
    ],
    compiler_params=pltpu.CompilerParams(use_tc_tiling_on_sc=False),
)
def _sc_degree(dst2d, out, didx, ones_v, acc, sem_i, sem_s):
    c = lax.axis_index("c")
    s = lax.axis_index("s")
    wid = s * _NC + c

    def zfill(j, carry):
        ones_v[pl.ds(j * 16, 16)] = jnp.zeros((16,), jnp.float32)
        return carry

    lax.fori_loop(0, _RNODE // 16, zfill, 0)
    pltpu.sync_copy(ones_v, acc.at[pl.ds(s * _RNODE, _RNODE)])
    for j in range(128 // 16):
        ones_v[pl.ds(j * 16, 16)] = jnp.ones((16,), jnp.float32)
    plsc.subcore_barrier()

    base = wid * _RPT
    ones_s = ones_v.at[pl.ds(0, _C)]

    idx_d = [None] * (_NOUT + 1)
    sc_d = [None] * _NOUT
    idx_d[0] = pltpu.async_copy(dst2d.at[pl.ds(base, _K), :], didx[0], sem_i)
    for i in range(_NOUT):
        ib = didx[i % 3]
        if i >= 2:
            for d in sc_d[i - 2]:
                d.wait()
        idx_d[i].wait()
        if i + 1 < _NOUT:
            idx_d[i + 1] = pltpu.async_copy(
                dst2d.at[pl.ds(base + (i + 1) * _K, _K), :],
                didx[(i + 1) % 3], sem_i)
        sc_d[i] = [pltpu.async_copy(ones_s, acc.at[ib.at[j]], sem_s,
                                    add=True)
                   for j in range(_K)]
    for i in (_NOUT - 2, _NOUT - 1):
        for d in sc_d[i]:
            d.wait()

    plsc.subcore_barrier()
    pltpu.sync_copy(acc.at[pl.ds(s * _RNODE, _RNODE)],
                    out.at[pl.ds(c * _NP + s * _RNODE, _RNODE)])


@functools.partial(
    pl.kernel,
    out_type=jax.ShapeDtypeStruct((_NC * _NP, _DH), jnp.float32),
    mesh=_mesh,
    scratch_types=[
        [pltpu.VMEM((2, _K, _C), jnp.int32)] * 3,
        [pltpu.VMEM((_K, _C, _DH), jnp.float32)] * 2,
        pltpu.VMEM((80, _DH), jnp.float32),
        pltpu.VMEM_SHARED((_NP, _DH), jnp.float32),
        pltpu.SemaphoreType.DMA,
        pltpu.SemaphoreType.DMA,
        pltpu.SemaphoreType.DMA,
    ],
    compiler_params=pltpu.CompilerParams(use_tc_tiling_on_sc=False),
)
def _sc_message(src2d, dst2d, table, out, ibuf, rows, zbuf, acc,
                sem_i, sem_g, sem_s):
    c = lax.axis_index("c")
    s = lax.axis_index("s")
    wid = s * _NC + c

    def zfill(k, carry):
        zbuf[k, pl.ds(0, 16)] = jnp.zeros((16,), jnp.float32)
        zbuf[k, pl.ds(16, 16)] = jnp.zeros((16,), jnp.float32)
        return carry

    lax.fori_loop(0, 80, zfill, 0)
    for t in range(_RNODE // 80):
        pltpu.sync_copy(zbuf, acc.at[pl.ds(s * _RNODE + t * 80, 80), :])
    plsc.subcore_barrier()

    base = wid * _RPT

    def idx_start(i, buf):
        return [pltpu.async_copy(src2d.at[pl.ds(base + i * _K, _K), :],
                                 buf.at[0], sem_i),
                pltpu.async_copy(dst2d.at[pl.ds(base + i * _K, _K), :],
                                 buf.at[1], sem_i)]

    idx_d = [None] * (_NOUT + 1)
    sc_d = [None] * _NOUT
    idx_d[0] = idx_start(0, ibuf[0])
    for i in range(_NOUT):
        rb = rows[i % 2]
        ib = ibuf[i % 3]
        if i >= 2:
            for d in sc_d[i - 2]:
                d.wait()
        for d in idx_d[i]:
            d.wait()
        if i + 1 < _NOUT:
            idx_d[i + 1] = idx_start(i + 1, ibuf[(i + 1) % 3])
        gs = [pltpu.async_copy(table.at[ib.at[0, j]], rb.at[j], sem_g)
              for j in range(_K)]
        for g in gs:
            g.wait()
        sc_d[i] = [pltpu.async_copy(rb.at[j], acc.at[ib.at[1, j]], sem_s,
                                    add=True)
                   for j in range(_K)]
    for i in (_NOUT - 2, _NOUT - 1):
        for d in sc_d[i]:
            d.wait()

    plsc.subcore_barrier()
    pltpu.sync_copy(acc.at[pl.ds(s * _RNODE, _RNODE), :],
                    out.at[pl.ds(c * _NP + s * _RNODE, _RNODE), :])


_BLK = 2048
_GRID = _NP // _BLK


def _tc1_body(d0_ref, d1_ref, x_ref, w1_ref, xwp_ref, dinv_ref):
    deg = d0_ref[...] + d1_ref[...] + 1.0
    dinv = lax.rsqrt(deg)
    dinv_ref[...] = dinv
    xw = jnp.dot(x_ref[...], w1_ref[...])
    xwp_ref[...] = dinv[:, None] * xw


def _tc1(d0, d1, xp, W1):
    return pl.pallas_call(
        _tc1_body,
        grid=(_GRID,),
        in_specs=[
            pl.BlockSpec((_BLK,), lambda i: (i,)),
            pl.BlockSpec((_BLK,), lambda i: (i,)),
            pl.BlockSpec((_BLK, _DIN), lambda i: (i, 0)),
            pl.BlockSpec((_DIN, _DH), lambda i: (0, 0)),
        ],
        out_specs=[
            pl.BlockSpec((_BLK, _DH), lambda i: (i, 0)),
            pl.BlockSpec((_BLK,), lambda i: (i,)),
        ],
        out_shape=[
            jax.ShapeDtypeStruct((_NP, _DH), jnp.float32),
            jax.ShapeDtypeStruct((_NP,), jnp.float32),
        ],
    )(d0, d1, xp, W1)


def _tc2_body(a0_ref, a1_ref, xwp_ref, dinv_ref, b1_ref, w2_ref, out_ref):
    dinv = dinv_ref[...][:, None]
    h = dinv * (a0_ref[...] + a1_ref[...] + xwp_ref[...]) + b1_ref[...]
    h = jnp.maximum(h, 0.0)
    hw = jnp.dot(h, w2_ref[...])
    out_ref[...] = dinv * hw


def _tc2(a0, a1, xwp, dinv, b1, W2):
    return pl.pallas_call(
        _tc2_body,
        grid=(_GRID,),
        in_specs=[
            pl.BlockSpec((_BLK, _DH), lambda i: (i, 0)),
            pl.BlockSpec((_BLK, _DH), lambda i: (i, 0)),
            pl.BlockSpec((_BLK, _DH), lambda i: (i, 0)),
            pl.BlockSpec((_BLK,), lambda i: (i,)),
            pl.BlockSpec((1, _DH), lambda i: (0, 0)),
            pl.BlockSpec((_DH, _DH), lambda i: (0, 0)),
        ],
        out_specs=pl.BlockSpec((_BLK, _DH), lambda i: (i, 0)),
        out_shape=jax.ShapeDtypeStruct((_NP, _DH), jnp.float32),
    )(a0, a1, xwp, dinv, b1, W2)


def _tc3_body(a0_ref, a1_ref, xwp_ref, dinv_ref, b2_ref, bi_ref,
              wf1_ref, bf1_ref, wf2_ref, bf2_ref, out_ref):
    h = (dinv_ref[...][:, None] * (a0_ref[...] + a1_ref[...] + xwp_ref[...])
         + b2_ref[...])
    h = jnp.maximum(h, 0.0)
    gids = lax.broadcasted_iota(jnp.int32, (_NP, _G), 1)
    onehot = (bi_ref[...] == gids).astype(jnp.float32)
    dn = (((0,), (0,)), ((), ()))
    ssum = lax.dot_general(onehot, h, dn, precision=lax.Precision.HIGHEST)
    cnt = lax.dot_general(onehot, jnp.ones((_NP, 1), jnp.float32), dn,
                          precision=lax.Precision.HIGHEST)
    gemb = ssum / jnp.maximum(cnt, 1.0)
    z = jnp.dot(gemb, wf1_ref[...]) + bf1_ref[...]
    z = jnp.maximum(z, 0.0)
    out_ref[...] = jnp.dot(z, wf2_ref[...]) + bf2_ref[...]


def _tc3(a0, a1, xwp, dinv, b2, bi2d, Wf1, bf1, Wf2, bf2):
    return pl.pallas_call(
        _tc3_body,
        out_shape=jax.ShapeDtypeStruct((_G, _DOUT), jnp.float32),
    )(a0, a1, xwp, dinv, b2, bi2d, Wf1, bf1, Wf2, bf2)


@jax.jit
def kernel(x, edge_index, batch_index, W1, b1, W2, b2, Wf1, bf1, Wf2, bf2):
    f32 = jnp.float32
    src2d = edge_index[0].reshape(_EROWS, _C)
    dst2d = edge_index[1].reshape(_EROWS, _C)

    pad = _NP - _N
    xp = jnp.concatenate([x, jnp.zeros((pad, _DIN), f32)], axis=0)
    bip = jnp.concatenate(
        [batch_index, jnp.full((pad,), _G, batch_index.dtype)]).reshape(_NP, 1)

    degp = _sc_degree(dst2d)

    xwp1, dinv = _tc1(degp[:_NP], degp[_NP:], xp, W1)

    acc1 = _sc_message(src2d, dst2d, xwp1)
    xwp2 = _tc2(acc1[:_NP], acc1[_NP:], xwp1, dinv, b1.reshape(1, _DH), W2)

    acc2 = _sc_message(src2d, dst2d, xwp2)
    out = _tc3(acc2[:_NP], acc2[_NP:], xwp2, dinv, b2.reshape(1, _DH), bip,
               Wf1, bf1.reshape(1, _DFC), Wf2, bf2.reshape(1, _DOUT))
    return out
